# baseline (speedup 1.0000x reference)
import functools

import jax
import jax.numpy as jnp
from jax import lax
from jax.experimental import pallas as pl
from jax.experimental.pallas import tpu as pltpu
from jax.experimental.pallas import tpu_sc as plsc

_B = 64
_N = 32768
_K = 1000
_R = 16

_NC = 2
_NS = 16
_NW = _NC * _NS
_RW = _B // _NW
_ZC = 2048


def _topk_mask(d, w):
    a = jnp.abs(d)
    bits = jax.lax.bitcast_convert_type(a, jnp.int32)
    one = jnp.int32(1)

    def kth_body(i, cur):
        cand = jnp.bitwise_or(cur, jnp.left_shift(one, 30 - i))
        cnt = jnp.sum((bits >= cand).astype(jnp.int32), axis=1,
                      keepdims=True)
        return jnp.where(cnt >= _K, cand, cur)

    kth = jax.lax.fori_loop(0, 31, kth_body, jnp.zeros((_R, 1), jnp.int32))

    gt = bits > kth
    need = _K - jnp.sum(gt.astype(jnp.int32), axis=1, keepdims=True)
    eq = bits == kth
    idx = jax.lax.broadcasted_iota(jnp.int32, bits.shape, 1)

    def cut_body(i, cur):
        cand = jnp.bitwise_or(cur, jnp.left_shift(one, 15 - i))
        cnt = jnp.sum((eq & (idx < cand)).astype(jnp.int32), axis=1,
                      keepdims=True)
        return jnp.where(cnt <= need, cand, cur)

    cut = jax.lax.fori_loop(0, 16, cut_body, jnp.zeros((_R, 1), jnp.int32))

    sel = gt | (eq & (idx < cut))
    return w + sel.astype(jnp.float32)


def _tc_mask_kernel(d_ref, w_ref, o_ref):
    o_ref[...] = _topk_mask(d_ref[...], w_ref[...])


def _tc_true_branch(difference, weight):
    return pl.pallas_call(
        _tc_mask_kernel,
        grid=(_B // _R,),
        in_specs=[
            pl.BlockSpec((_R, _N), lambda i: (i, 0)),
            pl.BlockSpec((_R, _N), lambda i: (i, 0)),
        ],
        out_specs=pl.BlockSpec((_R, _N), lambda i: (i, 0)),
        out_shape=jax.ShapeDtypeStruct((_B, _N), jnp.float32),
    )(difference, weight)


def _sc_zero_kernel(o_hbm, buf, sem):
    wid = lax.axis_index("s") * _NC + lax.axis_index("c")
    base = wid * _RW

    def zbody(i, carry):
        buf[pl.ds(i * 16, 16)] = jnp.zeros((16,), jnp.float32)
        return carry

    jax.lax.fori_loop(0, _ZC // 16, zbody, 0)

    copies = [
        pltpu.make_async_copy(
            buf, o_hbm.at[base + r].at[pl.ds(c * _ZC, _ZC)], sem)
        for r in range(_RW)
        for c in range(_N // _ZC)
    ]
    for cp in copies:
        cp.start()
    for cp in copies:
        cp.wait()


def _sc_false_branch():
    mesh = plsc.VectorSubcoreMesh(core_axis_name="c", subcore_axis_name="s")
    k = functools.partial(
        pl.kernel,
        mesh=mesh,
        out_type=jax.ShapeDtypeStruct((_B, _N), jnp.float32),
        scratch_types=[
            pltpu.VMEM((_ZC,), jnp.float32),
            pltpu.SemaphoreType.DMA,
        ],
    )(_sc_zero_kernel)
    return k()


def kernel(difference, weight, epoch, iteration):
    del iteration
    epoch = jnp.asarray(epoch, jnp.int32)
    cond = (epoch > 1000) & (epoch < 18000) & (epoch % 200 == 0)
    return lax.cond(
        cond,
        lambda: _tc_true_branch(difference, weight),
        lambda: _sc_false_branch(),
    )


# speedup vs baseline: 5.8236x; 5.8236x over previous
import jax
import jax.numpy as jnp
from jax import lax
from jax.experimental import pallas as pl
from jax.experimental.pallas import tpu as pltpu

_B = 64
_N = 32768
_K = 1000
_R = 16
_Z = 8


def _topk_mask(d, w):
    a = jnp.abs(d)
    bits = jax.lax.bitcast_convert_type(a, jnp.int32)
    one = jnp.int32(1)

    def kth_body(i, cur):
        cand = jnp.bitwise_or(cur, jnp.left_shift(one, 30 - i))
        cnt = jnp.sum((bits >= cand).astype(jnp.int32), axis=1,
                      keepdims=True)
        return jnp.where(cnt >= _K, cand, cur)

    kth = jax.lax.fori_loop(0, 31, kth_body, jnp.zeros((_R, 1), jnp.int32))

    gt = bits > kth
    need = _K - jnp.sum(gt.astype(jnp.int32), axis=1, keepdims=True)
    eq = bits == kth
    idx = jax.lax.broadcasted_iota(jnp.int32, bits.shape, 1)

    def cut_body(i, cur):
        cand = jnp.bitwise_or(cur, jnp.left_shift(one, 15 - i))
        cnt = jnp.sum((eq & (idx < cand)).astype(jnp.int32), axis=1,
                      keepdims=True)
        return jnp.where(cnt <= need, cand, cur)

    cut = jax.lax.fori_loop(0, 16, cut_body, jnp.zeros((_R, 1), jnp.int32))

    sel = gt | (eq & (idx < cut))
    return w + sel.astype(jnp.float32)


def _mask_kernel(d_ref, w_ref, o_ref):
    o_ref[...] = _topk_mask(d_ref[...], w_ref[...])


def _zero_kernel(o_hbm, z_s, sems):
    z_s[...] = jnp.zeros((_Z, _N), jnp.float32)
    outs = [
        pltpu.make_async_copy(
            z_s, o_hbm.at[pl.ds(c * _Z, _Z)], sems.at[c])
        for c in range(_B // _Z)
    ]
    for cp in outs:
        cp.start()
    for cp in outs:
        cp.wait()


def _true_branch(difference, weight):
    return pl.pallas_call(
        _mask_kernel,
        grid=(_B // _R,),
        in_specs=[
            pl.BlockSpec((_R, _N), lambda i: (i, 0)),
            pl.BlockSpec((_R, _N), lambda i: (i, 0)),
        ],
        out_specs=pl.BlockSpec((_R, _N), lambda i: (i, 0)),
        out_shape=jax.ShapeDtypeStruct((_B, _N), jnp.float32),
    )(difference, weight)


def _false_branch():
    return pl.pallas_call(
        _zero_kernel,
        out_specs=pl.BlockSpec(memory_space=pl.ANY),
        out_shape=jax.ShapeDtypeStruct((_B, _N), jnp.float32),
        scratch_shapes=[
            pltpu.VMEM((_Z, _N), jnp.float32),
            pltpu.SemaphoreType.DMA((_B // _Z,)),
        ],
    )()


def kernel(difference, weight, epoch, iteration):
    del iteration
    epoch = jnp.asarray(epoch, jnp.int32)
    cond = (epoch > 1000) & (epoch < 18000) & (epoch % 200 == 0)
    return lax.cond(
        cond,
        lambda: _true_branch(difference, weight),
        lambda: _false_branch(),
    )
